# baseline (speedup 1.0000x reference)
import jax
import jax.numpy as jnp
from jax import lax
from jax.experimental import pallas as pl
from jax.experimental.pallas import tpu as pltpu
from jax.experimental.pallas import tpu_sc as plsc

N = 10000
E = 320000
H = 8
DH = 16
D = H * DH
DW = D + DH

NC = 2
NS = 16
NW = NC * NS
EPW = E // NW
CH = 40
NCHUNK = EPW // CH
PAIRS = NCHUNK // 2
NP = 10112
RPT = NP // NS
ROUNDS = [(r * CH, CH) for r in range(RPT // CH)] + [(RPT - RPT % CH, RPT % CH)]

BLK = 2000


def _dense_body(x_ref, wst_ref, wdt_ref, bs_ref, bd_ref, al_ref, ar_ref,
                pm_ref, fsb_ref, el_ref, er_ref):
    x = x_ref[...]
    fs = jnp.dot(x, wst_ref[...], preferred_element_type=jnp.float32) + bs_ref[...]
    fd = jnp.dot(x, wdt_ref[...], preferred_element_type=jnp.float32) + bd_ref[...]
    fsb_ref[...] = jnp.dot(fs, pm_ref[...],
                           preferred_element_type=jnp.float32).astype(jnp.bfloat16)
    el_ref[...] = jnp.dot(fs, al_ref[...], preferred_element_type=jnp.float32)
    er_ref[...] = jnp.dot(fd, ar_ref[...], preferred_element_type=jnp.float32)


def _dense_stage(feats, wst, wdt, bs, bd, alp, arp, pm):
    grid = (N // BLK,)
    full = lambda s: pl.BlockSpec(s, lambda i: (0, 0))
    return pl.pallas_call(
        _dense_body,
        grid=grid,
        in_specs=[
            pl.BlockSpec((BLK, D), lambda i: (i, 0)),
            full((D, D)), full((D, D)), full((1, D)), full((1, D)),
            full((D, DH)), full((D, DH)), full((D, D)),
        ],
        out_specs=[
            pl.BlockSpec((BLK, D), lambda i: (i, 0)),
            pl.BlockSpec((BLK, DH), lambda i: (i, 0)),
            pl.BlockSpec((BLK, DH), lambda i: (i, 0)),
        ],
        out_shape=[
            jax.ShapeDtypeStruct((N, D), jnp.bfloat16),
            jax.ShapeDtypeStruct((N, DH), jnp.float32),
            jax.ShapeDtypeStruct((N, DH), jnp.float32),
        ],
    )(feats, wst, wdt, bs, bd, alp, arp, pm)


def _sc_body(fsb_hbm, er_hbm, src_hbm, dst_hbm,
             out0, out1,
             src_all, dst_all, erg0, erg1, fb0, fb1, fg0, fg1,
             acc_sh, sem_g0, sem_g1, sem_c0, sem_c1):
    cid = lax.axis_index("c")
    sid = lax.axis_index("s")
    wid = sid * NC + cid

    erg = (erg0, erg1)
    fb = (fb0, fb1)
    fg = (fg0, fg1)
    sem_g = (sem_g0, sem_g1)
    sem_c = (sem_c0, sem_c1)

    pltpu.sync_copy(src_hbm.at[wid], src_all)
    pltpu.sync_copy(dst_hbm.at[wid], dst_all)

    zeros16 = jnp.zeros((16,), jnp.float32)

    def zrow(i, carry):
        for j in range(DW // 16):
            fg0[i, pl.ds(j * 16, 16)] = zeros16
        return carry

    lax.fori_loop(0, CH, zrow, 0)

    for off, sz in ROUNDS:
        base = sid * RPT + off
        pltpu.sync_copy(fg0.at[pl.ds(0, sz)], acc_sh.at[pl.ds(base, sz)])
    plsc.subcore_barrier()

    def issue_gather(c, bi):
        pltpu.async_copy(fsb_hbm.at[src_all.at[c]], fb[bi], sem_g[bi])
        pltpu.async_copy(er_hbm.at[dst_all.at[c]], erg[bi], sem_g[bi])

    def wait_gather(c, bi):
        pltpu.make_async_copy(fsb_hbm.at[src_all.at[c]], fb[bi], sem_g[bi]).wait()
        pltpu.make_async_copy(er_hbm.at[dst_all.at[c]], erg[bi], sem_g[bi]).wait()

    def issue_scatter(c, bi):
        pltpu.async_copy(fg[bi], acc_sh.at[dst_all.at[c]], sem_c[bi], add=True)

    def wait_scatter(c, bi):
        pltpu.make_async_copy(fg[bi], acc_sh.at[dst_all.at[c]], sem_c[bi]).wait()

    def compute(bi):
        erg_b, fb_b, fg_b = erg[bi], fb[bi], fg[bi]

        @plsc.parallel_loop(0, CH, unroll=4)
        def erow(e):
            el_e = lax.bitcast_convert_type(fb_b[e, pl.ds(D // 2, DH)],
                                            jnp.float32)
            v = el_e + erg_b[e, :]
            v = jnp.where(v > 0, v, v * 0.01)
            sv = jnp.exp(v)
            fg_b[e, pl.ds(D, DH)] = sv
            for g in range(H // 2):
                pw = fb_b[e, pl.ds(g * 16, 16)]
                a = lax.bitcast_convert_type(lax.shift_left(pw, 16), jnp.float32)
                b = lax.bitcast_convert_type(pw & jnp.int32(-65536), jnp.float32)
                fg_b[e, pl.ds(g * 32, 16)] = a * sv[2 * g]
                fg_b[e, pl.ds(g * 32 + 16, 16)] = b * sv[2 * g + 1]

    issue_gather(0, 0)

    def pair(k, carry):
        c0 = 2 * k
        @pl.when(k > 0)
        def _():
            wait_scatter(c0 - 1, 1)
        issue_gather(c0 + 1, 1)
        wait_gather(c0, 0)
        compute(0)
        issue_scatter(c0, 0)
        wait_scatter(c0, 0)

        @pl.when(k + 1 < PAIRS)
        def _():
            issue_gather(c0 + 2, 0)

        wait_gather(c0 + 1, 1)
        compute(1)
        issue_scatter(c0 + 1, 1)
        return carry

    lax.fori_loop(0, PAIRS, pair, 0)
    wait_scatter(NCHUNK - 1, 1)
    plsc.subcore_barrier()

    for off, sz in ROUNDS:
        base = sid * RPT + off
        pltpu.sync_copy(acc_sh.at[pl.ds(base, sz)], fg0.at[pl.ds(0, sz)])

        @pl.when(cid == 0)
        def _():
            pltpu.sync_copy(fg0.at[pl.ds(0, sz)], out0.at[pl.ds(base, sz)])

        @pl.when(cid == 1)
        def _():
            pltpu.sync_copy(fg0.at[pl.ds(0, sz)], out1.at[pl.ds(base, sz)])


_sc_stage = pl.kernel(
    _sc_body,
    out_type=[
        jax.ShapeDtypeStruct((NP, DW), jnp.float32),
        jax.ShapeDtypeStruct((NP, DW), jnp.float32),
    ],
    mesh=plsc.VectorSubcoreMesh(
        core_axis_name="c", subcore_axis_name="s", num_cores=NC, num_subcores=NS),
    compiler_params=pltpu.CompilerParams(use_tc_tiling_on_sc=False),
    scratch_types=[
        pltpu.VMEM((NCHUNK, CH), jnp.int32),
        pltpu.VMEM((NCHUNK, CH), jnp.int32),
        pltpu.VMEM((CH, DH), jnp.float32),
        pltpu.VMEM((CH, DH), jnp.float32),
        pltpu.VMEM((CH, D // 2 + DH), jnp.int32),
        pltpu.VMEM((CH, D // 2 + DH), jnp.int32),
        pltpu.VMEM((CH, DW), jnp.float32),
        pltpu.VMEM((CH, DW), jnp.float32),
        pltpu.VMEM_SHARED((NP, DW), jnp.float32),
        pltpu.SemaphoreType.DMA,
        pltpu.SemaphoreType.DMA,
        pltpu.SemaphoreType.DMA,
        pltpu.SemaphoreType.DMA,
    ],
)


def _norm_body(p0_ref, p1_ref, exp_ref, o_ref):
    p = p0_ref[...] + p1_ref[...]
    es = jnp.dot(p[:, D:DW], exp_ref[...], preferred_element_type=jnp.float32)
    o_ref[...] = p[:, :D] / es


def _norm_stage(p0, p1, expm):
    grid = (N // BLK,)
    return pl.pallas_call(
        _norm_body,
        grid=grid,
        in_specs=[
            pl.BlockSpec((BLK, DW), lambda i: (i, 0)),
            pl.BlockSpec((BLK, DW), lambda i: (i, 0)),
            pl.BlockSpec((DH, D), lambda i: (0, 0)),
        ],
        out_specs=pl.BlockSpec((BLK, D), lambda i: (i, 0)),
        out_shape=jax.ShapeDtypeStruct((N, D), jnp.float32),
    )(p0, p1, expm)


def kernel(feats, edge_index, W_src, b_src, W_dst, b_dst, attn_l, attn_r):
    src = edge_index[0].astype(jnp.int32).reshape(NW, NCHUNK, CH)
    dst = edge_index[1].astype(jnp.int32).reshape(NW, NCHUNK, CH)
    f32 = jnp.float32
    rows = jnp.arange(D)
    alp = jnp.zeros((D, DH), f32).at[rows, rows // DH].set(attn_l.reshape(-1))
    arp = jnp.zeros((D, DH), f32).at[rows, rows // DH].set(attn_r.reshape(-1))
    m = jnp.arange(D)
    logical = (m // 32) * 32 + (m % 2) * 16 + (m % 32) // 2
    pm = (rows[:, None] == logical[None, :]).astype(f32)
    fsb, el16, er16 = _dense_stage(
        feats, W_src.T, W_dst.T, b_src.reshape(1, D), b_dst.reshape(1, D),
        alp, arp, pm)
    fsb32 = jax.lax.bitcast_convert_type(fsb.reshape(N, D // 2, 2), jnp.int32)
    el32 = jax.lax.bitcast_convert_type(el16, jnp.int32)
    fsbx = jnp.concatenate([fsb32, el32], axis=1)
    p0, p1 = _sc_stage(fsbx, er16, src, dst)
    expm = (jnp.arange(DH)[:, None] == (jnp.arange(D)[None, :] // DH)).astype(f32)
    return _norm_stage(p0, p1, expm)


# speedup vs baseline: 1.2237x; 1.2237x over previous
import jax
import jax.numpy as jnp
from jax import lax
from jax.experimental import pallas as pl
from jax.experimental.pallas import tpu as pltpu
from jax.experimental.pallas import tpu_sc as plsc

N = 10000
E = 320000
H = 8
DH = 16
D = H * DH
DW = D + DH

NC = 2
NS = 16
NW = NC * NS
EPW = E // NW
CH = 80
NCHUNK = EPW // CH
QUADS = (NCHUNK - 1) // 4
NP = 10112
RPT = NP // NS
ROUNDS = [(r * CH, CH) for r in range(RPT // CH)] + [(RPT - RPT % CH, RPT % CH)]

BLK = 2000


def _dense_body(x_ref, wst_ref, wdt_ref, bs_ref, bd_ref, al_ref, ar_ref,
                fsx_ref, er_ref):
    x = x_ref[...]
    fs = jnp.dot(x, wst_ref[...], preferred_element_type=jnp.float32) + bs_ref[...]
    fd = jnp.dot(x, wdt_ref[...], preferred_element_type=jnp.float32) + bd_ref[...]
    fsx_ref[:, pl.ds(0, D)] = fs
    fsx_ref[:, pl.ds(D, DH)] = jnp.dot(fs, al_ref[...],
                                       preferred_element_type=jnp.float32)
    er_ref[...] = jnp.dot(fd, ar_ref[...], preferred_element_type=jnp.float32)


def _dense_stage(feats, wst, wdt, bs, bd, alp, arp):
    grid = (N // BLK,)
    full = lambda s: pl.BlockSpec(s, lambda i: (0, 0))
    return pl.pallas_call(
        _dense_body,
        grid=grid,
        in_specs=[
            pl.BlockSpec((BLK, D), lambda i: (i, 0)),
            full((D, D)), full((D, D)), full((1, D)), full((1, D)),
            full((D, DH)), full((D, DH)),
        ],
        out_specs=[
            pl.BlockSpec((BLK, DW), lambda i: (i, 0)),
            pl.BlockSpec((BLK, DH), lambda i: (i, 0)),
        ],
        out_shape=[
            jax.ShapeDtypeStruct((N, DW), jnp.float32),
            jax.ShapeDtypeStruct((N, DH), jnp.float32),
        ],
    )(feats, wst, wdt, bs, bd, alp, arp)


def _sc_body(fsx_hbm, er_hbm, src_hbm, dst_hbm,
             out0, out1,
             src_all, db0, db1, db2, db3, erg0, erg1, fg0, fg1,
             acc_sh, si0, si1, si2, si3, sem_g0, sem_g1, sem_c0, sem_c1):
    cid = lax.axis_index("c")
    sid = lax.axis_index("s")
    wid = sid * NC + cid

    db = (db0, db1, db2, db3)
    erg = (erg0, erg1)
    fg = (fg0, fg1)
    sem_i = (si0, si1, si2, si3)
    sem_g = (sem_g0, sem_g1)
    sem_c = (sem_c0, sem_c1)

    pltpu.sync_copy(src_hbm.at[wid], src_all)

    zeros16 = jnp.zeros((16,), jnp.float32)

    def zrow(i, carry):
        for j in range(DW // 16):
            fg0[i, pl.ds(j * 16, 16)] = zeros16
        return carry

    lax.fori_loop(0, CH, zrow, 0)

    for off, sz in ROUNDS:
        base = sid * RPT + off
        pltpu.sync_copy(fg0.at[pl.ds(0, sz)], acc_sh.at[pl.ds(base, sz)])
    plsc.subcore_barrier()

    def issue_dst(c, b4):
        pltpu.async_copy(dst_hbm.at[wid, c], db[b4], sem_i[b4])

    def wait_dst(c, b4):
        pltpu.make_async_copy(dst_hbm.at[wid, c], db[b4], sem_i[b4]).wait()

    def issue_gather(c, b2, b4):
        pltpu.async_copy(fsx_hbm.at[src_all.at[c]], fg[b2], sem_g[b2])
        pltpu.async_copy(er_hbm.at[db[b4]], erg[b2], sem_g[b2])

    def wait_gather(c, b2, b4):
        pltpu.make_async_copy(fsx_hbm.at[src_all.at[c]], fg[b2], sem_g[b2]).wait()
        pltpu.make_async_copy(er_hbm.at[db[b4]], erg[b2], sem_g[b2]).wait()

    def issue_scatter(c, b2, b4):
        pltpu.async_copy(fg[b2], acc_sh.at[db[b4]], sem_c[b2], add=True)

    def wait_scatter(c, b2, b4):
        pltpu.make_async_copy(fg[b2], acc_sh.at[db[b4]], sem_c[b2]).wait()

    def compute(b2):
        erg_b, fg_b = erg[b2], fg[b2]

        @plsc.parallel_loop(0, CH, unroll=4)
        def erow(e):
            v = fg_b[e, pl.ds(D, DH)] + erg_b[e, :]
            v = jnp.where(v > 0, v, v * 0.01)
            sv = jnp.exp(v)
            fg_b[e, pl.ds(D, DH)] = sv
            for h in range(H):
                fg_b[e, pl.ds(h * 16, 16)] = fg_b[e, pl.ds(h * 16, 16)] * sv[h]

    def chunk_body(c, k, j):
        b2, b4 = j % 2, j % 4
        nb2, nb4 = (j + 1) % 2, (j + 1) % 4
        if j == 0:
            @pl.when(k > 0)
            def _():
                wait_scatter(c - 1, nb2, (j + 3) % 4)
        else:
            wait_scatter(c - 1, nb2, (j + 3) % 4)

        @pl.when(c + 3 < NCHUNK)
        def _():
            issue_dst(c + 3, (j + 3) % 4)

        wait_dst(c + 1, nb4)
        issue_gather(c + 1, nb2, nb4)
        wait_gather(c, b2, b4)
        compute(b2)
        issue_scatter(c, b2, b4)

    issue_dst(0, 0)
    issue_dst(1, 1)
    issue_dst(2, 2)
    wait_dst(0, 0)
    issue_gather(0, 0, 0)

    def quad(k, carry):
        for j in range(4):
            chunk_body(4 * k + j, k, j)
        return carry

    lax.fori_loop(0, QUADS, quad, 0)
    wait_scatter(NCHUNK - 2, 1, 3)
    wait_gather(NCHUNK - 1, 0, 0)
    compute(0)
    issue_scatter(NCHUNK - 1, 0, 0)
    wait_scatter(NCHUNK - 1, 0, 0)
    plsc.subcore_barrier()

    for off, sz in ROUNDS:
        base = sid * RPT + off
        pltpu.sync_copy(acc_sh.at[pl.ds(base, sz)], fg0.at[pl.ds(0, sz)])

        @pl.when(cid == 0)
        def _():
            pltpu.sync_copy(fg0.at[pl.ds(0, sz)], out0.at[pl.ds(base, sz)])

        @pl.when(cid == 1)
        def _():
            pltpu.sync_copy(fg0.at[pl.ds(0, sz)], out1.at[pl.ds(base, sz)])


_sc_stage = pl.kernel(
    _sc_body,
    out_type=[
        jax.ShapeDtypeStruct((NP, DW), jnp.float32),
        jax.ShapeDtypeStruct((NP, DW), jnp.float32),
    ],
    mesh=plsc.VectorSubcoreMesh(
        core_axis_name="c", subcore_axis_name="s", num_cores=NC, num_subcores=NS),
    compiler_params=pltpu.CompilerParams(use_tc_tiling_on_sc=False),
    scratch_types=[
        pltpu.VMEM((NCHUNK, CH), jnp.int32),
        pltpu.VMEM((CH,), jnp.int32),
        pltpu.VMEM((CH,), jnp.int32),
        pltpu.VMEM((CH,), jnp.int32),
        pltpu.VMEM((CH,), jnp.int32),
        pltpu.VMEM((CH, DH), jnp.float32),
        pltpu.VMEM((CH, DH), jnp.float32),
        pltpu.VMEM((CH, DW), jnp.float32),
        pltpu.VMEM((CH, DW), jnp.float32),
        pltpu.VMEM_SHARED((NP, DW), jnp.float32),
        pltpu.SemaphoreType.DMA,
        pltpu.SemaphoreType.DMA,
        pltpu.SemaphoreType.DMA,
        pltpu.SemaphoreType.DMA,
        pltpu.SemaphoreType.DMA,
        pltpu.SemaphoreType.DMA,
        pltpu.SemaphoreType.DMA,
        pltpu.SemaphoreType.DMA,
    ],
)


def _norm_body(p0_ref, p1_ref, exp_ref, o_ref):
    p = p0_ref[...] + p1_ref[...]
    es = jnp.dot(p[:, D:DW], exp_ref[...], preferred_element_type=jnp.float32)
    o_ref[...] = p[:, :D] / es


def _norm_stage(p0, p1, expm):
    grid = (N // BLK,)
    return pl.pallas_call(
        _norm_body,
        grid=grid,
        in_specs=[
            pl.BlockSpec((BLK, DW), lambda i: (i, 0)),
            pl.BlockSpec((BLK, DW), lambda i: (i, 0)),
            pl.BlockSpec((DH, D), lambda i: (0, 0)),
        ],
        out_specs=pl.BlockSpec((BLK, D), lambda i: (i, 0)),
        out_shape=jax.ShapeDtypeStruct((N, D), jnp.float32),
    )(p0, p1, expm)


def kernel(feats, edge_index, W_src, b_src, W_dst, b_dst, attn_l, attn_r):
    src = edge_index[0].astype(jnp.int32).reshape(NW, NCHUNK, CH)
    dst = edge_index[1].astype(jnp.int32).reshape(NW, NCHUNK, CH)
    f32 = jnp.float32
    rows = jnp.arange(D)
    alp = jnp.zeros((D, DH), f32).at[rows, rows // DH].set(attn_l.reshape(-1))
    arp = jnp.zeros((D, DH), f32).at[rows, rows // DH].set(attn_r.reshape(-1))
    fsx, er16 = _dense_stage(
        feats, W_src.T, W_dst.T, b_src.reshape(1, D), b_dst.reshape(1, D),
        alp, arp)
    p0, p1 = _sc_stage(fsx, er16, src, dst)
    expm = (jnp.arange(DH)[:, None] == (jnp.arange(D)[None, :] // DH)).astype(f32)
    return _norm_stage(p0, p1, expm)


# speedup vs baseline: 1.2246x; 1.0008x over previous
import jax
import jax.numpy as jnp
from jax import lax
from jax.experimental import pallas as pl
from jax.experimental.pallas import tpu as pltpu
from jax.experimental.pallas import tpu_sc as plsc

N = 10000
E = 320000
H = 8
DH = 16
D = H * DH
DW = D + DH

NC = 2
NS = 16
NW = NC * NS
EPW = E // NW
CH = 80
NCHUNK = EPW // CH
QUADS = (NCHUNK - 1) // 4
NP = 10112
RPT = NP // NS
ROUNDS = [(r * CH, CH) for r in range(RPT // CH)] + [(RPT - RPT % CH, RPT % CH)]

BLK = 2000


def _dense_body(x_ref, wst_ref, wdt_ref, bs_ref, bd_ref, al_ref, ar_ref,
                fsx_ref, er_ref):
    x = x_ref[...]
    fs = jnp.dot(x, wst_ref[...], preferred_element_type=jnp.float32) + bs_ref[...]
    fd = jnp.dot(x, wdt_ref[...], preferred_element_type=jnp.float32) + bd_ref[...]
    fsx_ref[:, pl.ds(0, D)] = fs
    fsx_ref[:, pl.ds(D, DH)] = jnp.dot(fs, al_ref[...],
                                       preferred_element_type=jnp.float32)
    er_ref[...] = jnp.dot(fd, ar_ref[...], preferred_element_type=jnp.float32)


def _dense_stage(feats, wst, wdt, bs, bd, alp, arp):
    grid = (N // BLK,)
    full = lambda s: pl.BlockSpec(s, lambda i: (0, 0))
    return pl.pallas_call(
        _dense_body,
        grid=grid,
        in_specs=[
            pl.BlockSpec((BLK, D), lambda i: (i, 0)),
            full((D, D)), full((D, D)), full((1, D)), full((1, D)),
            full((D, DH)), full((D, DH)),
        ],
        out_specs=[
            pl.BlockSpec((BLK, DW), lambda i: (i, 0)),
            pl.BlockSpec((BLK, DH), lambda i: (i, 0)),
        ],
        out_shape=[
            jax.ShapeDtypeStruct((N, DW), jnp.float32),
            jax.ShapeDtypeStruct((N, DH), jnp.float32),
        ],
    )(feats, wst, wdt, bs, bd, alp, arp)


def _sc_body(fsx_hbm, er_hbm, src_hbm, dst_hbm,
             out0, out1,
             src_all, db0, db1, db2, db3, erg0, erg1, fg0, fg1,
             acc_sh, si0, si1, si2, si3, sem_g0, sem_g1, sem_c0, sem_c1):
    cid = lax.axis_index("c")
    sid = lax.axis_index("s")
    wid = sid * NC + cid

    db = (db0, db1, db2, db3)
    erg = (erg0, erg1)
    fg = (fg0, fg1)
    sem_i = (si0, si1, si2, si3)
    sem_g = (sem_g0, sem_g1)
    sem_c = (sem_c0, sem_c1)

    pltpu.sync_copy(src_hbm.at[wid], src_all)

    zeros16 = jnp.zeros((16,), jnp.float32)

    def zrow(i, carry):
        for j in range(DW // 16):
            fg0[i, pl.ds(j * 16, 16)] = zeros16
        return carry

    lax.fori_loop(0, CH, zrow, 0)

    for off, sz in ROUNDS:
        base = sid * RPT + off
        pltpu.sync_copy(fg0.at[pl.ds(0, sz)], acc_sh.at[pl.ds(base, sz)])
    plsc.subcore_barrier()

    def issue_dst(c, b4):
        pltpu.async_copy(dst_hbm.at[wid, c], db[b4], sem_i[b4])

    def wait_dst(c, b4):
        pltpu.make_async_copy(dst_hbm.at[wid, c], db[b4], sem_i[b4]).wait()

    def issue_gather(c, b2, b4):
        pltpu.async_copy(fsx_hbm.at[src_all.at[c]], fg[b2], sem_g[b2])
        pltpu.async_copy(er_hbm.at[db[b4]], erg[b2], sem_g[b2])

    def wait_gather(c, b2, b4):
        pltpu.make_async_copy(fsx_hbm.at[src_all.at[c]], fg[b2], sem_g[b2]).wait()
        pltpu.make_async_copy(er_hbm.at[db[b4]], erg[b2], sem_g[b2]).wait()

    def issue_scatter(c, b2, b4):
        pltpu.async_copy(fg[b2], acc_sh.at[db[b4]], sem_c[b2], add=True)

    def wait_scatter(c, b2, b4):
        pltpu.make_async_copy(fg[b2], acc_sh.at[db[b4]], sem_c[b2]).wait()

    def compute(b2):
        erg_b, fg_b = erg[b2], fg[b2]

        @plsc.parallel_loop(0, CH, unroll=2)
        def erow(e):
            v = fg_b[e, pl.ds(D, DH)] + erg_b[e, :]
            v = jnp.where(v > 0, v, v * 0.01)
            sv = jnp.exp(v)
            fg_b[e, pl.ds(D, DH)] = sv
            for h in range(H):
                fg_b[e, pl.ds(h * 16, 16)] = fg_b[e, pl.ds(h * 16, 16)] * sv[h]

    def chunk_body(c, k, j):
        b2, b4 = j % 2, j % 4
        nb2, nb4 = (j + 1) % 2, (j + 1) % 4
        if j == 0:
            @pl.when(k > 0)
            def _():
                wait_scatter(c - 1, nb2, (j + 3) % 4)
        else:
            wait_scatter(c - 1, nb2, (j + 3) % 4)

        @pl.when(c + 3 < NCHUNK)
        def _():
            issue_dst(c + 3, (j + 3) % 4)

        wait_dst(c + 1, nb4)
        issue_gather(c + 1, nb2, nb4)
        wait_gather(c, b2, b4)
        compute(b2)
        issue_scatter(c, b2, b4)

    issue_dst(0, 0)
    issue_dst(1, 1)
    issue_dst(2, 2)
    wait_dst(0, 0)
    issue_gather(0, 0, 0)

    def quad(k, carry):
        for j in range(4):
            chunk_body(4 * k + j, k, j)
        return carry

    lax.fori_loop(0, QUADS, quad, 0)
    wait_scatter(NCHUNK - 2, 1, 3)
    wait_gather(NCHUNK - 1, 0, 0)
    compute(0)
    issue_scatter(NCHUNK - 1, 0, 0)
    wait_scatter(NCHUNK - 1, 0, 0)
    plsc.subcore_barrier()

    for off, sz in ROUNDS:
        base = sid * RPT + off
        pltpu.sync_copy(acc_sh.at[pl.ds(base, sz)], fg0.at[pl.ds(0, sz)])

        @pl.when(cid == 0)
        def _():
            pltpu.sync_copy(fg0.at[pl.ds(0, sz)], out0.at[pl.ds(base, sz)])

        @pl.when(cid == 1)
        def _():
            pltpu.sync_copy(fg0.at[pl.ds(0, sz)], out1.at[pl.ds(base, sz)])


_sc_stage = pl.kernel(
    _sc_body,
    out_type=[
        jax.ShapeDtypeStruct((NP, DW), jnp.float32),
        jax.ShapeDtypeStruct((NP, DW), jnp.float32),
    ],
    mesh=plsc.VectorSubcoreMesh(
        core_axis_name="c", subcore_axis_name="s", num_cores=NC, num_subcores=NS),
    compiler_params=pltpu.CompilerParams(use_tc_tiling_on_sc=False),
    scratch_types=[
        pltpu.VMEM((NCHUNK, CH), jnp.int32),
        pltpu.VMEM((CH,), jnp.int32),
        pltpu.VMEM((CH,), jnp.int32),
        pltpu.VMEM((CH,), jnp.int32),
        pltpu.VMEM((CH,), jnp.int32),
        pltpu.VMEM((CH, DH), jnp.float32),
        pltpu.VMEM((CH, DH), jnp.float32),
        pltpu.VMEM((CH, DW), jnp.float32),
        pltpu.VMEM((CH, DW), jnp.float32),
        pltpu.VMEM_SHARED((NP, DW), jnp.float32),
        pltpu.SemaphoreType.DMA,
        pltpu.SemaphoreType.DMA,
        pltpu.SemaphoreType.DMA,
        pltpu.SemaphoreType.DMA,
        pltpu.SemaphoreType.DMA,
        pltpu.SemaphoreType.DMA,
        pltpu.SemaphoreType.DMA,
        pltpu.SemaphoreType.DMA,
    ],
)


def _norm_body(p0_ref, p1_ref, exp_ref, o_ref):
    p = p0_ref[...] + p1_ref[...]
    es = jnp.dot(p[:, D:DW], exp_ref[...], preferred_element_type=jnp.float32)
    o_ref[...] = p[:, :D] / es


def _norm_stage(p0, p1, expm):
    grid = (N // BLK,)
    return pl.pallas_call(
        _norm_body,
        grid=grid,
        in_specs=[
            pl.BlockSpec((BLK, DW), lambda i: (i, 0)),
            pl.BlockSpec((BLK, DW), lambda i: (i, 0)),
            pl.BlockSpec((DH, D), lambda i: (0, 0)),
        ],
        out_specs=pl.BlockSpec((BLK, D), lambda i: (i, 0)),
        out_shape=jax.ShapeDtypeStruct((N, D), jnp.float32),
    )(p0, p1, expm)


def kernel(feats, edge_index, W_src, b_src, W_dst, b_dst, attn_l, attn_r):
    src = edge_index[0].astype(jnp.int32).reshape(NW, NCHUNK, CH)
    dst = edge_index[1].astype(jnp.int32).reshape(NW, NCHUNK, CH)
    f32 = jnp.float32
    rows = jnp.arange(D)
    alp = jnp.zeros((D, DH), f32).at[rows, rows // DH].set(attn_l.reshape(-1))
    arp = jnp.zeros((D, DH), f32).at[rows, rows // DH].set(attn_r.reshape(-1))
    fsx, er16 = _dense_stage(
        feats, W_src.T, W_dst.T, b_src.reshape(1, D), b_dst.reshape(1, D),
        alp, arp)
    p0, p1 = _sc_stage(fsx, er16, src, dst)
    expm = (jnp.arange(DH)[:, None] == (jnp.arange(D)[None, :] // DH)).astype(f32)
    return _norm_stage(p0, p1, expm)


# speedup vs baseline: 1.2482x; 1.0193x over previous
import jax
import jax.numpy as jnp
from jax import lax
from jax.experimental import pallas as pl
from jax.experimental.pallas import tpu as pltpu
from jax.experimental.pallas import tpu_sc as plsc

N = 10000
E = 320000
H = 8
DH = 16
D = H * DH
DW = D + DH

NC = 2
NS = 16
NW = NC * NS
EPW = E // NW
CH = 80
NCHUNK = EPW // CH
QUADS = (NCHUNK - 1) // 4
NP = 10112
RPT = NP // NS
ROUNDS = [(r * CH, CH) for r in range(RPT // CH)] + [(RPT - RPT % CH, RPT % CH)]

BLK = 2000


def _dense_body(x_ref, wst_ref, wdt_ref, bs_ref, bd_ref, al_ref, ar_ref,
                fsx_ref, er_ref):
    x = x_ref[...]
    fs = jnp.dot(x, wst_ref[...], preferred_element_type=jnp.float32) + bs_ref[...]
    fd = jnp.dot(x, wdt_ref[...], preferred_element_type=jnp.float32) + bd_ref[...]
    fsx_ref[:, pl.ds(0, D)] = fs
    fsx_ref[:, pl.ds(D, DH)] = jnp.dot(fs, al_ref[...],
                                       preferred_element_type=jnp.float32)
    er_ref[...] = jnp.dot(fd, ar_ref[...], preferred_element_type=jnp.float32)


def _dense_stage(feats, wst, wdt, bs, bd, alp, arp):
    grid = (N // BLK,)
    full = lambda s: pl.BlockSpec(s, lambda i: (0, 0))
    return pl.pallas_call(
        _dense_body,
        grid=grid,
        in_specs=[
            pl.BlockSpec((BLK, D), lambda i: (i, 0)),
            full((D, D)), full((D, D)), full((1, D)), full((1, D)),
            full((D, DH)), full((D, DH)),
        ],
        out_specs=[
            pl.BlockSpec((BLK, DW), lambda i: (i, 0)),
            pl.BlockSpec((BLK, DH), lambda i: (i, 0)),
        ],
        out_shape=[
            jax.ShapeDtypeStruct((N, DW), jnp.float32),
            jax.ShapeDtypeStruct((N, DH), jnp.float32),
        ],
    )(feats, wst, wdt, bs, bd, alp, arp)


def _sc_body(fsx_hbm, er_hbm, src_hbm, dst_hbm,
             out0, out1,
             src_all, db0, db1, db2, db3, erg0, erg1, fg0, fg1,
             acc_sh, si0, si1, si2, si3, sem_g0, sem_g1, sem_c0, sem_c1):
    cid = lax.axis_index("c")
    sid = lax.axis_index("s")
    wid = sid * NC + cid

    db = (db0, db1, db2, db3)
    erg = (erg0, erg1)
    fg = (fg0, fg1)
    sem_i = (si0, si1, si2, si3)
    sem_g = (sem_g0, sem_g1)
    sem_c = (sem_c0, sem_c1)

    pltpu.sync_copy(src_hbm.at[wid], src_all)

    zeros16 = jnp.zeros((16,), jnp.float32)

    def zrow(i, carry):
        for j in range(DW // 16):
            fg0[i, pl.ds(j * 16, 16)] = zeros16
        return carry

    lax.fori_loop(0, CH, zrow, 0)

    for off, sz in ROUNDS:
        base = sid * RPT + off
        pltpu.sync_copy(fg0.at[pl.ds(0, sz)], acc_sh.at[pl.ds(base, sz)])
    plsc.subcore_barrier()

    def issue_dst(c, b4):
        pltpu.async_copy(dst_hbm.at[wid, c], db[b4], sem_i[b4])

    def wait_dst(c, b4):
        pltpu.make_async_copy(dst_hbm.at[wid, c], db[b4], sem_i[b4]).wait()

    HF = CH // 2

    def issue_gather(c, b2, b4):
        pltpu.async_copy(fsx_hbm.at[src_all.at[c]], fg[b2], sem_g[b2])
        pltpu.async_copy(er_hbm.at[db[b4].at[0]], erg[b2].at[pl.ds(0, HF)],
                         sem_g[b2])
        pltpu.async_copy(er_hbm.at[db[b4].at[1]], erg[b2].at[pl.ds(HF, HF)],
                         sem_g[b2])

    def wait_gather(c, b2, b4):
        pltpu.make_async_copy(fsx_hbm.at[src_all.at[c]], fg[b2], sem_g[b2]).wait()
        pltpu.make_async_copy(er_hbm.at[db[b4].at[0]], erg[b2].at[pl.ds(0, HF)],
                              sem_g[b2]).wait()
        pltpu.make_async_copy(er_hbm.at[db[b4].at[1]], erg[b2].at[pl.ds(HF, HF)],
                              sem_g[b2]).wait()

    def issue_scatter_half(c, b2, b4, half):
        pltpu.async_copy(fg[b2].at[pl.ds(half * HF, HF)],
                         acc_sh.at[db[b4].at[half]], sem_c[b2], add=True)

    def wait_scatter(c, b2, b4):
        pltpu.make_async_copy(fg[b2].at[pl.ds(0, HF)],
                              acc_sh.at[db[b4].at[0]], sem_c[b2]).wait()
        pltpu.make_async_copy(fg[b2].at[pl.ds(HF, HF)],
                              acc_sh.at[db[b4].at[1]], sem_c[b2]).wait()

    def compute(b2, lo):
        erg_b, fg_b = erg[b2], fg[b2]

        @plsc.parallel_loop(lo, lo + HF, unroll=2)
        def erow(e):
            v = fg_b[e, pl.ds(D, DH)] + erg_b[e, :]
            v = jnp.where(v > 0, v, v * 0.01)
            sv = jnp.exp(v)
            fg_b[e, pl.ds(D, DH)] = sv
            for h in range(H):
                fg_b[e, pl.ds(h * 16, 16)] = fg_b[e, pl.ds(h * 16, 16)] * sv[h]

    def chunk_body(c, k, j):
        b2, b4 = j % 2, j % 4
        nb2, nb4 = (j + 1) % 2, (j + 1) % 4
        if j == 0:
            @pl.when(k > 0)
            def _():
                wait_scatter(c - 1, nb2, (j + 3) % 4)
        else:
            wait_scatter(c - 1, nb2, (j + 3) % 4)

        @pl.when(c + 3 < NCHUNK)
        def _():
            issue_dst(c + 3, (j + 3) % 4)

        wait_dst(c + 1, nb4)
        issue_gather(c + 1, nb2, nb4)
        wait_gather(c, b2, b4)
        compute(b2, 0)
        issue_scatter_half(c, b2, b4, 0)
        compute(b2, HF)
        issue_scatter_half(c, b2, b4, 1)

    issue_dst(0, 0)
    issue_dst(1, 1)
    issue_dst(2, 2)
    wait_dst(0, 0)
    issue_gather(0, 0, 0)

    def quad(k, carry):
        for j in range(4):
            chunk_body(4 * k + j, k, j)
        return carry

    lax.fori_loop(0, QUADS, quad, 0)
    wait_scatter(NCHUNK - 2, 1, 3)
    wait_gather(NCHUNK - 1, 0, 0)
    compute(0, 0)
    issue_scatter_half(NCHUNK - 1, 0, 0, 0)
    compute(0, HF)
    issue_scatter_half(NCHUNK - 1, 0, 0, 1)
    wait_scatter(NCHUNK - 1, 0, 0)
    plsc.subcore_barrier()

    for off, sz in ROUNDS:
        base = sid * RPT + off
        pltpu.sync_copy(acc_sh.at[pl.ds(base, sz)], fg0.at[pl.ds(0, sz)])

        @pl.when(cid == 0)
        def _():
            pltpu.sync_copy(fg0.at[pl.ds(0, sz)], out0.at[pl.ds(base, sz)])

        @pl.when(cid == 1)
        def _():
            pltpu.sync_copy(fg0.at[pl.ds(0, sz)], out1.at[pl.ds(base, sz)])


_sc_stage = pl.kernel(
    _sc_body,
    out_type=[
        jax.ShapeDtypeStruct((NP, DW), jnp.float32),
        jax.ShapeDtypeStruct((NP, DW), jnp.float32),
    ],
    mesh=plsc.VectorSubcoreMesh(
        core_axis_name="c", subcore_axis_name="s", num_cores=NC, num_subcores=NS),
    compiler_params=pltpu.CompilerParams(use_tc_tiling_on_sc=False),
    scratch_types=[
        pltpu.VMEM((NCHUNK, CH), jnp.int32),
        pltpu.VMEM((2, CH // 2), jnp.int32),
        pltpu.VMEM((2, CH // 2), jnp.int32),
        pltpu.VMEM((2, CH // 2), jnp.int32),
        pltpu.VMEM((2, CH // 2), jnp.int32),
        pltpu.VMEM((CH, DH), jnp.float32),
        pltpu.VMEM((CH, DH), jnp.float32),
        pltpu.VMEM((CH, DW), jnp.float32),
        pltpu.VMEM((CH, DW), jnp.float32),
        pltpu.VMEM_SHARED((NP, DW), jnp.float32),
        pltpu.SemaphoreType.DMA,
        pltpu.SemaphoreType.DMA,
        pltpu.SemaphoreType.DMA,
        pltpu.SemaphoreType.DMA,
        pltpu.SemaphoreType.DMA,
        pltpu.SemaphoreType.DMA,
        pltpu.SemaphoreType.DMA,
        pltpu.SemaphoreType.DMA,
    ],
)


def _norm_body(p0_ref, p1_ref, exp_ref, o_ref):
    p = p0_ref[...] + p1_ref[...]
    es = jnp.dot(p[:, D:DW], exp_ref[...], preferred_element_type=jnp.float32)
    o_ref[...] = p[:, :D] / es


def _norm_stage(p0, p1, expm):
    grid = (N // BLK,)
    return pl.pallas_call(
        _norm_body,
        grid=grid,
        in_specs=[
            pl.BlockSpec((BLK, DW), lambda i: (i, 0)),
            pl.BlockSpec((BLK, DW), lambda i: (i, 0)),
            pl.BlockSpec((DH, D), lambda i: (0, 0)),
        ],
        out_specs=pl.BlockSpec((BLK, D), lambda i: (i, 0)),
        out_shape=jax.ShapeDtypeStruct((N, D), jnp.float32),
    )(p0, p1, expm)


def kernel(feats, edge_index, W_src, b_src, W_dst, b_dst, attn_l, attn_r):
    src = edge_index[0].astype(jnp.int32).reshape(NW, NCHUNK, CH)
    dst = edge_index[1].astype(jnp.int32).reshape(NW, NCHUNK, 2, CH // 2)
    f32 = jnp.float32
    rows = jnp.arange(D)
    alp = jnp.zeros((D, DH), f32).at[rows, rows // DH].set(attn_l.reshape(-1))
    arp = jnp.zeros((D, DH), f32).at[rows, rows // DH].set(attn_r.reshape(-1))
    fsx, er16 = _dense_stage(
        feats, W_src.T, W_dst.T, b_src.reshape(1, D), b_dst.reshape(1, D),
        alp, arp)
    p0, p1 = _sc_stage(fsx, er16, src, dst)
    expm = (jnp.arange(DH)[:, None] == (jnp.arange(D)[None, :] // DH)).astype(f32)
    return _norm_stage(p0, p1, expm)


# speedup vs baseline: 1.2482x; 1.0000x over previous
import jax
import jax.numpy as jnp
from jax import lax
from jax.experimental import pallas as pl
from jax.experimental.pallas import tpu as pltpu
from jax.experimental.pallas import tpu_sc as plsc

N = 10000
E = 320000
H = 8
DH = 16
D = H * DH
DW = D + DH

NC = 2
NS = 16
NW = NC * NS
EPW = E // NW
CH = 80
NCHUNK = EPW // CH
QUADS = (NCHUNK - 1) // 4
NP = 10112
RPT = NP // NS
ROUNDS = [(r * CH, CH) for r in range(RPT // CH)] + [(RPT - RPT % CH, RPT % CH)]

BLK = 2000


def _dense_body(x_ref, wst_ref, wdt_ref, bs_ref, bd_ref, al_ref, ar_ref,
                fsx_ref, er_ref):
    x = x_ref[...]
    fs = jnp.dot(x, wst_ref[...], preferred_element_type=jnp.float32) + bs_ref[...]
    fd = jnp.dot(x, wdt_ref[...], preferred_element_type=jnp.float32) + bd_ref[...]
    fsx_ref[:, pl.ds(0, D)] = fs
    fsx_ref[:, pl.ds(D, DH)] = jnp.dot(fs, al_ref[...],
                                       preferred_element_type=jnp.float32)
    er_ref[...] = jnp.dot(fd, ar_ref[...], preferred_element_type=jnp.float32)


def _dense_stage(feats, wst, wdt, bs, bd, alp, arp):
    grid = (N // BLK,)
    full = lambda s: pl.BlockSpec(s, lambda i: (0, 0))
    return pl.pallas_call(
        _dense_body,
        grid=grid,
        in_specs=[
            pl.BlockSpec((BLK, D), lambda i: (i, 0)),
            full((D, D)), full((D, D)), full((1, D)), full((1, D)),
            full((D, DH)), full((D, DH)),
        ],
        out_specs=[
            pl.BlockSpec((BLK, DW), lambda i: (i, 0)),
            pl.BlockSpec((BLK, DH), lambda i: (i, 0)),
        ],
        out_shape=[
            jax.ShapeDtypeStruct((N, DW), jnp.float32),
            jax.ShapeDtypeStruct((N, DH), jnp.float32),
        ],
    )(feats, wst, wdt, bs, bd, alp, arp)


def _sc_body(fsx_hbm, er_hbm, src_hbm, dst_hbm,
             out0, out1,
             src_all, db0, db1, db2, db3, erg0, erg1, fg0, fg1,
             acc_sh, si0, si1, si2, si3, sem_g0, sem_g1, sem_c0, sem_c1):
    cid = lax.axis_index("c")
    sid = lax.axis_index("s")
    wid = sid * NC + cid

    db = (db0, db1, db2, db3)
    erg = (erg0, erg1)
    fg = (fg0, fg1)
    sem_i = (si0, si1, si2, si3)
    sem_g = (sem_g0, sem_g1)
    sem_c = (sem_c0, sem_c1)

    pltpu.sync_copy(src_hbm.at[wid], src_all)

    zeros16 = jnp.zeros((16,), jnp.float32)

    def zrow(i, carry):
        for j in range(DW // 16):
            fg0[i, pl.ds(j * 16, 16)] = zeros16
        return carry

    lax.fori_loop(0, CH, zrow, 0)

    for off, sz in ROUNDS:
        base = sid * RPT + off
        pltpu.sync_copy(fg0.at[pl.ds(0, sz)], acc_sh.at[pl.ds(base, sz)])
    plsc.subcore_barrier()

    def issue_dst(c, b4):
        pltpu.async_copy(dst_hbm.at[wid, c], db[b4], sem_i[b4])

    def wait_dst(c, b4):
        pltpu.make_async_copy(dst_hbm.at[wid, c], db[b4], sem_i[b4]).wait()

    HF = CH // 2

    def issue_gather(c, b2, b4):
        pltpu.async_copy(fsx_hbm.at[src_all.at[c]], fg[b2], sem_g[b2])
        pltpu.async_copy(er_hbm.at[db[b4].at[0]], erg[b2].at[pl.ds(0, HF)],
                         sem_g[b2])
        pltpu.async_copy(er_hbm.at[db[b4].at[1]], erg[b2].at[pl.ds(HF, HF)],
                         sem_g[b2])

    def wait_gather(c, b2, b4):
        pltpu.make_async_copy(fsx_hbm.at[src_all.at[c]], fg[b2], sem_g[b2]).wait()
        pltpu.make_async_copy(er_hbm.at[db[b4].at[0]], erg[b2].at[pl.ds(0, HF)],
                              sem_g[b2]).wait()
        pltpu.make_async_copy(er_hbm.at[db[b4].at[1]], erg[b2].at[pl.ds(HF, HF)],
                              sem_g[b2]).wait()

    def issue_scatter_half(c, b2, b4, half):
        pltpu.async_copy(fg[b2].at[pl.ds(half * HF, HF)],
                         acc_sh.at[db[b4].at[half]], sem_c[b2], add=True)

    def wait_scatter(c, b2, b4):
        pltpu.make_async_copy(fg[b2].at[pl.ds(0, HF)],
                              acc_sh.at[db[b4].at[0]], sem_c[b2]).wait()
        pltpu.make_async_copy(fg[b2].at[pl.ds(HF, HF)],
                              acc_sh.at[db[b4].at[1]], sem_c[b2]).wait()

    def compute(b2, lo):
        erg_b, fg_b = erg[b2], fg[b2]

        @plsc.parallel_loop(lo, lo + HF, unroll=2)
        def erow(e):
            v = fg_b[e, pl.ds(D, DH)] + erg_b[e, :]
            v = jnp.where(v > 0, v, v * 0.01)
            sv = jnp.exp(v)
            fg_b[e, pl.ds(D, DH)] = sv
            for h in range(H):
                fg_b[e, pl.ds(h * 16, 16)] = fg_b[e, pl.ds(h * 16, 16)] * sv[h]

    def chunk_body(c, k, j):
        b2, b4 = j % 2, j % 4
        nb2, nb4 = (j + 1) % 2, (j + 1) % 4
        if j == 0:
            @pl.when(k > 0)
            def _():
                wait_scatter(c - 1, nb2, (j + 3) % 4)
        else:
            wait_scatter(c - 1, nb2, (j + 3) % 4)

        @pl.when(c + 3 < NCHUNK)
        def _():
            issue_dst(c + 3, (j + 3) % 4)

        wait_dst(c + 1, nb4)
        issue_gather(c + 1, nb2, nb4)
        wait_gather(c, b2, b4)
        compute(b2, 0)
        issue_scatter_half(c, b2, b4, 0)
        compute(b2, HF)
        issue_scatter_half(c, b2, b4, 1)

    issue_dst(0, 0)
    issue_dst(1, 1)
    issue_dst(2, 2)
    wait_dst(0, 0)
    issue_gather(0, 0, 0)

    def quad(k, carry):
        for j in range(4):
            chunk_body(4 * k + j, k, j)
        return carry

    lax.fori_loop(0, QUADS, quad, 0)
    wait_scatter(NCHUNK - 2, 1, 3)
    wait_gather(NCHUNK - 1, 0, 0)
    compute(0, 0)
    issue_scatter_half(NCHUNK - 1, 0, 0, 0)
    compute(0, HF)
    issue_scatter_half(NCHUNK - 1, 0, 0, 1)
    wait_scatter(NCHUNK - 1, 0, 0)
    plsc.subcore_barrier()

    for off, sz in ROUNDS:
        base = sid * RPT + off
        pltpu.sync_copy(acc_sh.at[pl.ds(base, sz)], fg0.at[pl.ds(0, sz)])

        @pl.when(cid == 0)
        def _():
            pltpu.sync_copy(fg0.at[pl.ds(0, sz)], out0.at[pl.ds(base, sz)])

        @pl.when(cid == 1)
        def _():
            pltpu.sync_copy(fg0.at[pl.ds(0, sz)], out1.at[pl.ds(base, sz)])


def _make_sc_stage():
    return pl.kernel(
        _sc_body,
    out_type=[
        jax.ShapeDtypeStruct((NP, DW), jnp.float32),
        jax.ShapeDtypeStruct((NP, DW), jnp.float32),
    ],
    mesh=plsc.VectorSubcoreMesh(
        core_axis_name="c", subcore_axis_name="s", num_cores=NC, num_subcores=NS),
    compiler_params=pltpu.CompilerParams(use_tc_tiling_on_sc=False),
    scratch_types=[
        pltpu.VMEM((NCHUNK, CH), jnp.int32),
        pltpu.VMEM((2, CH // 2), jnp.int32),
        pltpu.VMEM((2, CH // 2), jnp.int32),
        pltpu.VMEM((2, CH // 2), jnp.int32),
        pltpu.VMEM((2, CH // 2), jnp.int32),
        pltpu.VMEM((CH, DH), jnp.float32),
        pltpu.VMEM((CH, DH), jnp.float32),
        pltpu.VMEM((CH, DW), jnp.float32),
        pltpu.VMEM((CH, DW), jnp.float32),
        pltpu.VMEM_SHARED((NP, DW), jnp.float32),
        pltpu.SemaphoreType.DMA,
        pltpu.SemaphoreType.DMA,
        pltpu.SemaphoreType.DMA,
        pltpu.SemaphoreType.DMA,
        pltpu.SemaphoreType.DMA,
        pltpu.SemaphoreType.DMA,
        pltpu.SemaphoreType.DMA,
        pltpu.SemaphoreType.DMA,
    ],
)


def _norm_body(p0_ref, p1_ref, exp_ref, o_ref):
    p = p0_ref[...] + p1_ref[...]
    es = jnp.dot(p[:, D:DW], exp_ref[...], preferred_element_type=jnp.float32)
    o_ref[...] = p[:, :D] / es


def _norm_stage(p0, p1, expm):
    grid = (N // BLK,)
    return pl.pallas_call(
        _norm_body,
        grid=grid,
        in_specs=[
            pl.BlockSpec((BLK, DW), lambda i: (i, 0)),
            pl.BlockSpec((BLK, DW), lambda i: (i, 0)),
            pl.BlockSpec((DH, D), lambda i: (0, 0)),
        ],
        out_specs=pl.BlockSpec((BLK, D), lambda i: (i, 0)),
        out_shape=jax.ShapeDtypeStruct((N, D), jnp.float32),
    )(p0, p1, expm)


def kernel(feats, edge_index, W_src, b_src, W_dst, b_dst, attn_l, attn_r):
    src = edge_index[0].astype(jnp.int32).reshape(NW, NCHUNK, CH)
    dst = edge_index[1].astype(jnp.int32).reshape(NW, NCHUNK, 2, CH // 2)
    f32 = jnp.float32
    rows = jnp.arange(D)
    alp = jnp.zeros((D, DH), f32).at[rows, rows // DH].set(attn_l.reshape(-1))
    arp = jnp.zeros((D, DH), f32).at[rows, rows // DH].set(attn_r.reshape(-1))
    fsx, er16 = _dense_stage(
        feats, W_src.T, W_dst.T, b_src.reshape(1, D), b_dst.reshape(1, D),
        alp, arp)
    p0, p1 = _make_sc_stage()(fsx, er16, src, dst)
    expm = (jnp.arange(DH)[:, None] == (jnp.arange(D)[None, :] // DH)).astype(f32)
    return _norm_stage(p0, p1, expm)
